# Initial kernel scaffold; baseline (speedup 1.0000x reference)
import functools

import jax
import jax.numpy as jnp
from jax import lax
from jax.experimental import pallas as pl
from jax.experimental.pallas import tpu as pltpu
from jax.experimental.pallas import tpu_sc as plsc

N = 10000
E = 320000
D = 128
H = 128
G = 64
EPS = 1e-5

NC = 2
NS = 16
EB = 128

E_PER_SUB = E // NS
N_FULL = E_PER_SUB // EB
E_REM = E_PER_SUB - N_FULL * EB
ROWS_PER_SUB = N // NS
DEG_W = 16

_MESH = plsc.VectorSubcoreMesh(
    core_axis_name="c", subcore_axis_name="s", num_cores=NC, num_subcores=NS
)


@functools.partial(
    pl.kernel,
    out_type=jax.ShapeDtypeStruct((NC, N, DEG_W), jnp.float32),
    mesh=_MESH,
    scratch_types=[
        pltpu.VMEM((EB, DEG_W), jnp.float32),
        pltpu.VMEM((EB,), jnp.int32),
        pltpu.VMEM((E_REM,), jnp.int32),
        pltpu.VMEM_SHARED((N, DEG_W), jnp.float32),
    ],
)
def _deg_sc(dst_hbm, zeros_hbm, out_hbm, ones_v, idx_v, idx_r, acc):
    c = lax.axis_index("c")
    s = lax.axis_index("s")

    def fill_ones(r, carry):
        ones_v[r, :] = jnp.ones((DEG_W,), jnp.float32)
        return carry

    lax.fori_loop(0, EB, fill_ones, 0)
    pltpu.sync_copy(zeros_hbm, acc.at[pl.ds(s * ROWS_PER_SUB, ROWS_PER_SUB)])
    plsc.subcore_barrier()

    base = s * E_PER_SUB

    def body(j, carry):
        pltpu.sync_copy(dst_hbm.at[c, pl.ds(base + j * EB, EB)], idx_v)
        pltpu.sync_copy(ones_v, acc.at[idx_v], add=True)
        return carry

    lax.fori_loop(0, N_FULL, body, 0)
    pltpu.sync_copy(dst_hbm.at[c, pl.ds(base + N_FULL * EB, E_REM)], idx_r)
    pltpu.sync_copy(ones_v.at[pl.ds(0, E_REM)], acc.at[idx_r], add=True)

    plsc.subcore_barrier()
    pltpu.sync_copy(
        acc.at[pl.ds(s * ROWS_PER_SUB, ROWS_PER_SUB)],
        out_hbm.at[c, pl.ds(s * ROWS_PER_SUB, ROWS_PER_SUB)],
    )


@functools.partial(
    pl.kernel,
    out_type=jax.ShapeDtypeStruct((NC, N, H), jnp.float32),
    mesh=_MESH,
    scratch_types=[
        pltpu.VMEM((EB,), jnp.int32),
        pltpu.VMEM((EB,), jnp.int32),
        pltpu.VMEM((E_REM,), jnp.int32),
        pltpu.VMEM((E_REM,), jnp.int32),
        pltpu.VMEM((EB, H), jnp.float32),
        pltpu.VMEM_SHARED((N, H), jnp.float32),
        pltpu.SemaphoreType.DMA,
    ],
)
def _agg_sc(hp_hbm, src_hbm, dst_hbm, zeros_hbm, out_hbm,
            src_v, dst_v, src_r, dst_r, rows_v, acc, sem):
    c = lax.axis_index("c")
    s = lax.axis_index("s")

    pltpu.sync_copy(zeros_hbm, acc.at[pl.ds(s * ROWS_PER_SUB, ROWS_PER_SUB)])
    plsc.subcore_barrier()

    base = s * E_PER_SUB

    def body(j, carry):
        pltpu.sync_copy(src_hbm.at[c, pl.ds(base + j * EB, EB)], src_v)
        pltpu.sync_copy(dst_hbm.at[c, pl.ds(base + j * EB, EB)], dst_v)
        pltpu.async_copy(hp_hbm.at[src_v], rows_v, sem).wait()
        pltpu.sync_copy(rows_v, acc.at[dst_v], add=True)
        return carry

    lax.fori_loop(0, N_FULL, body, 0)
    rem = base + N_FULL * EB
    pltpu.sync_copy(src_hbm.at[c, pl.ds(rem, E_REM)], src_r)
    pltpu.sync_copy(dst_hbm.at[c, pl.ds(rem, E_REM)], dst_r)
    pltpu.async_copy(hp_hbm.at[src_r], rows_v.at[pl.ds(0, E_REM)], sem).wait()
    pltpu.sync_copy(rows_v.at[pl.ds(0, E_REM)], acc.at[dst_r], add=True)

    plsc.subcore_barrier()
    pltpu.sync_copy(
        acc.at[pl.ds(s * ROWS_PER_SUB, ROWS_PER_SUB)],
        out_hbm.at[c, pl.ds(s * ROWS_PER_SUB, ROWS_PER_SUB)],
    )


def _front_tc(x_ref, deg_ref, gamma_ref, beta_ref, w1_ref, b1_ref, w2_ref,
              b2_ref, wc1_ref, h1p_ref, dinv_ref):
    x = x_ref[0]
    mean = jnp.mean(x, axis=0, keepdims=True)
    xc = x - mean
    var = jnp.mean(xc * xc, axis=0, keepdims=True)
    xn = xc / jnp.sqrt(var + EPS) * gamma_ref[...] + beta_ref[...]
    h = jnp.maximum(jnp.dot(xn, w1_ref[...],
                            preferred_element_type=jnp.float32) + b1_ref[...], 0.0)
    h = jnp.maximum(jnp.dot(h, w2_ref[...],
                            preferred_element_type=jnp.float32) + b2_ref[...], 0.0)
    deg = deg_ref[0, :, 0:1] + 1.0
    dinv = 1.0 / jnp.sqrt(deg)
    dinv_ref[0] = dinv
    h1 = jnp.dot(h, wc1_ref[...], preferred_element_type=jnp.float32)
    h1p_ref[...] = h1 * dinv


def _epi_tc(agg_ref, hp_ref, dinv_ref, bc_ref, wn_ref, out_ref):
    dinv = dinv_ref[0]
    x = jnp.maximum((agg_ref[0] + hp_ref[...]) * dinv + bc_ref[...], 0.0)
    out_ref[...] = jnp.dot(x, wn_ref[...],
                           preferred_element_type=jnp.float32) * dinv


def _pool_tc(agg_ref, hp_ref, dinv_ref, bc_ref, batch_ref, out_ref):
    dinv = dinv_ref[0]
    x = jnp.maximum((agg_ref[0] + hp_ref[...]) * dinv + bc_ref[...], 0.0)
    b = batch_ref[0]
    gids = lax.broadcasted_iota(jnp.int32, (G, N), 0)
    onehot = (b == gids).astype(jnp.float32)
    ssum = jnp.dot(onehot, x, preferred_element_type=jnp.float32)
    cnt = jnp.sum(onehot, axis=1, keepdims=True)
    out_ref[0] = ssum / jnp.maximum(cnt, 1.0)


def _head_tc(p_ref, w3_ref, b3_ref, w4_ref, b4_ref, w5_ref, b5_ref, out_ref):
    h = jnp.concatenate([p_ref[0], p_ref[1]], axis=1)
    h = jnp.maximum(jnp.dot(h, w3_ref[...],
                            preferred_element_type=jnp.float32) + b3_ref[...], 0.0)
    h = jnp.maximum(jnp.dot(h, w4_ref[...],
                            preferred_element_type=jnp.float32) + b4_ref[...], 0.0)
    out_ref[...] = jnp.dot(h, w5_ref[...],
                           preferred_element_type=jnp.float32) + b5_ref[...]


def _row(x):
    return x.reshape(1, -1)


def kernel(x0, x1, edge_index0, edge_index1, batch0, batch1, gamma, beta,
           W1, b1, W2, b2, Wc1, bc1, Wc2, bc2, Wc3, bc3, W3, b3, W4, b4,
           W5, b5):
    xs = jnp.stack([x0, x1])
    src = jnp.stack([edge_index0[0], edge_index1[0] + N]).astype(jnp.int32)
    dst = jnp.stack([edge_index0[1], edge_index1[1]]).astype(jnp.int32)
    batch = jnp.stack([batch0, batch1]).astype(jnp.int32).reshape(NC, 1, N)
    zeros_h = jnp.zeros((ROWS_PER_SUB, H), jnp.float32)
    zeros_d = jnp.zeros((ROWS_PER_SUB, DEG_W), jnp.float32)

    deg_raw = _deg_sc(dst, zeros_d)

    front = pl.pallas_call(
        _front_tc,
        grid=(NC,),
        in_specs=[
            pl.BlockSpec((1, N, D), lambda b: (b, 0, 0)),
            pl.BlockSpec((1, N, DEG_W), lambda b: (b, 0, 0)),
            pl.BlockSpec((1, D), lambda b: (0, 0)),
            pl.BlockSpec((1, D), lambda b: (0, 0)),
            pl.BlockSpec((D, 2 * H), lambda b: (0, 0)),
            pl.BlockSpec((1, 2 * H), lambda b: (0, 0)),
            pl.BlockSpec((2 * H, H), lambda b: (0, 0)),
            pl.BlockSpec((1, H), lambda b: (0, 0)),
            pl.BlockSpec((H, H), lambda b: (0, 0)),
        ],
        out_specs=[
            pl.BlockSpec((N, H), lambda b: (b, 0)),
            pl.BlockSpec((1, N, 1), lambda b: (b, 0, 0)),
        ],
        out_shape=[
            jax.ShapeDtypeStruct((NC * N, H), jnp.float32),
            jax.ShapeDtypeStruct((NC, N, 1), jnp.float32),
        ],
    )
    h1p, dinv = front(xs, deg_raw, _row(gamma), _row(beta), W1, _row(b1),
                      W2, _row(b2), Wc1)

    def conv_step(hp, bc, w_next):
        agg = _agg_sc(hp, src, dst, zeros_h)
        epi = pl.pallas_call(
            _epi_tc,
            grid=(NC,),
            in_specs=[
                pl.BlockSpec((1, N, H), lambda b: (b, 0, 0)),
                pl.BlockSpec((N, H), lambda b: (b, 0)),
                pl.BlockSpec((1, N, 1), lambda b: (b, 0, 0)),
                pl.BlockSpec((1, H), lambda b: (0, 0)),
                pl.BlockSpec((H, H), lambda b: (0, 0)),
            ],
            out_specs=pl.BlockSpec((N, H), lambda b: (b, 0)),
            out_shape=jax.ShapeDtypeStruct((NC * N, H), jnp.float32),
        )
        return epi(agg, hp, dinv, _row(bc), w_next)

    h2p = conv_step(h1p, bc1, Wc2)
    h3p = conv_step(h2p, bc2, Wc3)

    agg3 = _agg_sc(h3p, src, dst, zeros_h)
    pool = pl.pallas_call(
        _pool_tc,
        grid=(NC,),
        in_specs=[
            pl.BlockSpec((1, N, H), lambda b: (b, 0, 0)),
            pl.BlockSpec((N, H), lambda b: (b, 0)),
            pl.BlockSpec((1, N, 1), lambda b: (b, 0, 0)),
            pl.BlockSpec((1, H), lambda b: (0, 0)),
            pl.BlockSpec((1, 1, N), lambda b: (b, 0, 0)),
        ],
        out_specs=pl.BlockSpec((1, G, H), lambda b: (b, 0, 0)),
        out_shape=jax.ShapeDtypeStruct((NC, G, H), jnp.float32),
    )
    pooled = pool(agg3, h3p, dinv, _row(bc3), batch)

    head = pl.pallas_call(
        _head_tc,
        out_shape=jax.ShapeDtypeStruct((G, 1), jnp.float32),
    )
    return head(pooled, W3, _row(b3), W4, _row(b4), W5, _row(b5))


# speedup vs baseline: 11.2335x; 11.2335x over previous
import functools

import jax
import jax.numpy as jnp
from jax import lax
from jax.experimental import pallas as pl
from jax.experimental.pallas import tpu as pltpu
from jax.experimental.pallas import tpu_sc as plsc

N = 10000
E = 320000
D = 128
H = 128
G = 64
EPS = 1e-5

NC = 2
NS = 16
EB = 128

NB = E // EB
NB_BASE = NB // NS
NB_EXTRA = NB - NB_BASE * NS
ROWS_PER_SUB = 624
ROWS_TAIL = N - ROWS_PER_SUB * NS
DEG_W = 16

_MESH = plsc.VectorSubcoreMesh(
    core_axis_name="c", subcore_axis_name="s", num_cores=NC, num_subcores=NS
)


def _chunk(c, s):
    start = s * NB_BASE + jnp.minimum(s, NB_EXTRA)
    nb = NB_BASE + (s < NB_EXTRA).astype(jnp.int32)
    return c * E + start * EB, nb


def _zero_acc(zeros_hbm, acc, s):
    pltpu.sync_copy(
        zeros_hbm.at[pl.ds(0, ROWS_PER_SUB)],
        acc.at[pl.ds(s * ROWS_PER_SUB, ROWS_PER_SUB)],
    )
    @pl.when(s == 0)
    def _():
        pltpu.sync_copy(
            zeros_hbm.at[pl.ds(0, ROWS_TAIL)],
            acc.at[pl.ds(NS * ROWS_PER_SUB, ROWS_TAIL)],
        )


def _write_out(acc, out_hbm, c, s):
    pltpu.sync_copy(
        acc.at[pl.ds(s * ROWS_PER_SUB, ROWS_PER_SUB)],
        out_hbm.at[c, pl.ds(s * ROWS_PER_SUB, ROWS_PER_SUB)],
    )
    @pl.when(s == 0)
    def _():
        pltpu.sync_copy(
            acc.at[pl.ds(NS * ROWS_PER_SUB, ROWS_TAIL)],
            out_hbm.at[c, pl.ds(NS * ROWS_PER_SUB, ROWS_TAIL)],
        )


@functools.partial(
    pl.kernel,
    out_type=jax.ShapeDtypeStruct((NC, N, DEG_W), jnp.float32),
    mesh=_MESH,
    scratch_types=[
        pltpu.VMEM((EB, DEG_W), jnp.float32),
        pltpu.VMEM((EB,), jnp.int32),
        pltpu.VMEM_SHARED((N, DEG_W), jnp.float32),
    ],
)
def _deg_sc(dst_hbm, zeros_hbm, out_hbm, ones_v, idx_v, acc):
    c = lax.axis_index("c")
    s = lax.axis_index("s")

    def fill_ones(r, carry):
        ones_v[r, :] = jnp.ones((DEG_W,), jnp.float32)
        return carry

    lax.fori_loop(0, EB, fill_ones, 0)
    _zero_acc(zeros_hbm, acc, s)
    plsc.subcore_barrier()

    base, nb = _chunk(c, s)

    def body(j, carry):
        pltpu.sync_copy(dst_hbm.at[pl.ds(base + j * EB, EB)], idx_v)
        pltpu.sync_copy(ones_v, acc.at[idx_v], add=True)
        return carry

    lax.fori_loop(0, nb, body, 0)

    plsc.subcore_barrier()
    _write_out(acc, out_hbm, c, s)


@functools.partial(
    pl.kernel,
    out_type=jax.ShapeDtypeStruct((NC, N, H), jnp.float32),
    mesh=_MESH,
    scratch_types=[
        pltpu.VMEM((EB,), jnp.int32),
        pltpu.VMEM((EB,), jnp.int32),
        pltpu.VMEM((EB, H), jnp.float32),
        pltpu.VMEM_SHARED((N, H), jnp.float32),
        pltpu.SemaphoreType.DMA,
    ],
)
def _agg_sc(hp_hbm, src_hbm, dst_hbm, zeros_hbm, out_hbm,
            src_v, dst_v, rows_v, acc, sem):
    c = lax.axis_index("c")
    s = lax.axis_index("s")

    _zero_acc(zeros_hbm, acc, s)
    plsc.subcore_barrier()

    base, nb = _chunk(c, s)

    def body(j, carry):
        pltpu.sync_copy(src_hbm.at[pl.ds(base + j * EB, EB)], src_v)
        pltpu.sync_copy(dst_hbm.at[pl.ds(base + j * EB, EB)], dst_v)
        pltpu.async_copy(hp_hbm.at[src_v], rows_v, sem).wait()
        pltpu.sync_copy(rows_v, acc.at[dst_v], add=True)
        return carry

    lax.fori_loop(0, nb, body, 0)

    plsc.subcore_barrier()
    _write_out(acc, out_hbm, c, s)


def _front_tc(x_ref, deg_ref, gamma_ref, beta_ref, w1_ref, b1_ref, w2_ref,
              b2_ref, wc1_ref, h1p_ref, dinv_ref):
    x = x_ref[0]
    mean = jnp.mean(x, axis=0, keepdims=True)
    xc = x - mean
    var = jnp.mean(xc * xc, axis=0, keepdims=True)
    xn = xc / jnp.sqrt(var + EPS) * gamma_ref[...] + beta_ref[...]
    h = jnp.maximum(jnp.dot(xn, w1_ref[...],
                            preferred_element_type=jnp.float32) + b1_ref[...], 0.0)
    h = jnp.maximum(jnp.dot(h, w2_ref[...],
                            preferred_element_type=jnp.float32) + b2_ref[...], 0.0)
    deg = deg_ref[0, :, 0:1] + 1.0
    dinv = 1.0 / jnp.sqrt(deg)
    dinv_ref[0] = dinv
    h1 = jnp.dot(h, wc1_ref[...], preferred_element_type=jnp.float32)
    h1p_ref[...] = h1 * dinv


def _epi_tc(agg_ref, hp_ref, dinv_ref, bc_ref, wn_ref, out_ref):
    dinv = dinv_ref[0]
    x = jnp.maximum((agg_ref[0] + hp_ref[...]) * dinv + bc_ref[...], 0.0)
    out_ref[...] = jnp.dot(x, wn_ref[...],
                           preferred_element_type=jnp.float32) * dinv


def _pool_tc(agg_ref, hp_ref, dinv_ref, bc_ref, batch_ref, out_ref):
    dinv = dinv_ref[0]
    x = jnp.maximum((agg_ref[0] + hp_ref[...]) * dinv + bc_ref[...], 0.0)
    b = batch_ref[0]
    gids = lax.broadcasted_iota(jnp.int32, (G, N), 0)
    onehot = (b == gids).astype(jnp.float32)
    ssum = jnp.dot(onehot, x, preferred_element_type=jnp.float32)
    cnt = jnp.sum(onehot, axis=1, keepdims=True)
    out_ref[0] = ssum / jnp.maximum(cnt, 1.0)


def _head_tc(p_ref, w3_ref, b3_ref, w4_ref, b4_ref, w5_ref, b5_ref, out_ref):
    h = jnp.concatenate([p_ref[0], p_ref[1]], axis=1)
    h = jnp.maximum(jnp.dot(h, w3_ref[...],
                            preferred_element_type=jnp.float32) + b3_ref[...], 0.0)
    h = jnp.maximum(jnp.dot(h, w4_ref[...],
                            preferred_element_type=jnp.float32) + b4_ref[...], 0.0)
    out_ref[...] = jnp.dot(h, w5_ref[...],
                           preferred_element_type=jnp.float32) + b5_ref[...]


def _row(x):
    return x.reshape(1, -1)


def kernel(x0, x1, edge_index0, edge_index1, batch0, batch1, gamma, beta,
           W1, b1, W2, b2, Wc1, bc1, Wc2, bc2, Wc3, bc3, W3, b3, W4, b4,
           W5, b5):
    xs = jnp.stack([x0, x1])
    src = jnp.concatenate(
        [edge_index0[0], edge_index1[0] + N]).astype(jnp.int32)
    dst = jnp.concatenate(
        [edge_index0[1], edge_index1[1]]).astype(jnp.int32)
    batch = jnp.stack([batch0, batch1]).astype(jnp.int32).reshape(NC, 1, N)
    zeros_h = jnp.zeros((ROWS_PER_SUB, H), jnp.float32)
    zeros_d = jnp.zeros((ROWS_PER_SUB, DEG_W), jnp.float32)

    deg_raw = _deg_sc(dst, zeros_d)

    front = pl.pallas_call(
        _front_tc,
        grid=(NC,),
        in_specs=[
            pl.BlockSpec((1, N, D), lambda b: (b, 0, 0)),
            pl.BlockSpec((1, N, DEG_W), lambda b: (b, 0, 0)),
            pl.BlockSpec((1, D), lambda b: (0, 0)),
            pl.BlockSpec((1, D), lambda b: (0, 0)),
            pl.BlockSpec((D, 2 * H), lambda b: (0, 0)),
            pl.BlockSpec((1, 2 * H), lambda b: (0, 0)),
            pl.BlockSpec((2 * H, H), lambda b: (0, 0)),
            pl.BlockSpec((1, H), lambda b: (0, 0)),
            pl.BlockSpec((H, H), lambda b: (0, 0)),
        ],
        out_specs=[
            pl.BlockSpec((N, H), lambda b: (b, 0)),
            pl.BlockSpec((1, N, 1), lambda b: (b, 0, 0)),
        ],
        out_shape=[
            jax.ShapeDtypeStruct((NC * N, H), jnp.float32),
            jax.ShapeDtypeStruct((NC, N, 1), jnp.float32),
        ],
    )
    h1p, dinv = front(xs, deg_raw, _row(gamma), _row(beta), W1, _row(b1),
                      W2, _row(b2), Wc1)

    def conv_step(hp, bc, w_next):
        agg = _agg_sc(hp, src, dst, zeros_h)
        epi = pl.pallas_call(
            _epi_tc,
            grid=(NC,),
            in_specs=[
                pl.BlockSpec((1, N, H), lambda b: (b, 0, 0)),
                pl.BlockSpec((N, H), lambda b: (b, 0)),
                pl.BlockSpec((1, N, 1), lambda b: (b, 0, 0)),
                pl.BlockSpec((1, H), lambda b: (0, 0)),
                pl.BlockSpec((H, H), lambda b: (0, 0)),
            ],
            out_specs=pl.BlockSpec((N, H), lambda b: (b, 0)),
            out_shape=jax.ShapeDtypeStruct((NC * N, H), jnp.float32),
        )
        return epi(agg, hp, dinv, _row(bc), w_next)

    h2p = conv_step(h1p, bc1, Wc2)
    h3p = conv_step(h2p, bc2, Wc3)

    agg3 = _agg_sc(h3p, src, dst, zeros_h)
    pool = pl.pallas_call(
        _pool_tc,
        grid=(NC,),
        in_specs=[
            pl.BlockSpec((1, N, H), lambda b: (b, 0, 0)),
            pl.BlockSpec((N, H), lambda b: (b, 0)),
            pl.BlockSpec((1, N, 1), lambda b: (b, 0, 0)),
            pl.BlockSpec((1, H), lambda b: (0, 0)),
            pl.BlockSpec((1, 1, N), lambda b: (b, 0, 0)),
        ],
        out_specs=pl.BlockSpec((1, G, H), lambda b: (b, 0, 0)),
        out_shape=jax.ShapeDtypeStruct((NC, G, H), jnp.float32),
    )
    pooled = pool(agg3, h3p, dinv, _row(bc3), batch)

    head = pl.pallas_call(
        _head_tc,
        out_shape=jax.ShapeDtypeStruct((G, 1), jnp.float32),
    )
    return head(pooled, W3, _row(b3), W4, _row(b4), W5, _row(b5))
